# baseline (speedup 1.0000x reference)
import functools

import jax
import jax.numpy as jnp
from jax import lax
from jax.experimental import pallas as pl
from jax.experimental.pallas import tpu as pltpu
from jax.experimental.pallas import tpu_sc as plsc

CHUNK = 128
NUM_CORES = 2
NUM_SUBCORES = 16
NW = NUM_CORES * NUM_SUBCORES


def _sc_segment_sum(x, e3, acc_rows, n_chunks):
    n_nodes, feat = x.shape
    rpt = acc_rows // NUM_SUBCORES
    lanes = feat // 16

    mesh = plsc.VectorSubcoreMesh(core_axis_name="c", subcore_axis_name="s")

    @functools.partial(
        pl.kernel,
        mesh=mesh,
        out_type=jax.ShapeDtypeStruct((NUM_CORES, acc_rows, feat), jnp.float32),
        scratch_types=[
            pltpu.VMEM((n_chunks // 2, CHUNK), jnp.int32),
            pltpu.VMEM((n_chunks // 2, CHUNK), jnp.int32),
            [pltpu.VMEM((CHUNK, feat), jnp.float32) for _ in range(2)],
            pltpu.VMEM_SHARED((acc_rows, feat), jnp.float32),
            [pltpu.SemaphoreType.DMA for _ in range(2)],
            [pltpu.SemaphoreType.DMA for _ in range(2)],
        ],
    )
    def seg_sum(x_hbm, e3_hbm, out_hbm,
                src_v, dst_v, bufs, acc_sh, sem_g, sem_s):
        rows_v = bufs[0]
        c = lax.axis_index("c")
        s = lax.axis_index("s")
        wid = c * NUM_SUBCORES + s

        zrows = 120
        zero_v = jnp.zeros((16,), jnp.float32)

        def zero_row(r, carry):
            for l in range(lanes):
                rows_v[r, pl.ds(l * 16, 16)] = zero_v
            return carry

        lax.fori_loop(0, zrows, zero_row, 0)
        base = s * rpt
        off = 0
        while off < rpt:
            n = min(zrows, rpt - off)
            pltpu.sync_copy(rows_v.at[pl.ds(0, n)],
                            acc_sh.at[pl.ds(base + off, n)])
            off += n

        plsc.subcore_barrier()

        ph_chunks = n_chunks // 2
        n_pairs = ph_chunks // 2
        for phase in range(2):
            coff = wid * n_chunks + phase * ph_chunks
            pltpu.sync_copy(e3_hbm.at[0, pl.ds(coff, ph_chunks)], src_v)
            pltpu.sync_copy(e3_hbm.at[1, pl.ds(coff, ph_chunks)], dst_v)
            for b in range(2):
                pltpu.async_copy(x_hbm.at[src_v.at[b]], bufs[b], sem_g[b])

            def pair_body(i, carry):
                for b in range(2):
                    j = 2 * i + b
                    pltpu.make_async_copy(x_hbm.at[src_v.at[j]], bufs[b],
                                          sem_g[b]).wait()
                    pltpu.async_copy(bufs[b], acc_sh.at[dst_v.at[j]],
                                     sem_s[b], add=True)
                for b in range(2):
                    j = 2 * i + b
                    pltpu.make_async_copy(bufs[b], acc_sh.at[dst_v.at[j]],
                                          sem_s[b]).wait()

                    @pl.when(i < n_pairs - 1)
                    def _next_gather():
                        pltpu.async_copy(x_hbm.at[src_v.at[j + 2]], bufs[b],
                                         sem_g[b])
                return carry

            lax.fori_loop(0, n_pairs, pair_body, 0)
        plsc.subcore_barrier()
        pltpu.sync_copy(acc_sh.at[pl.ds(base, rpt)],
                        out_hbm.at[c, pl.ds(base, rpt)])

    return seg_sum(x, e3)


def _tc_finish(partials, x, w_cat, bias2d, blk):
    n_nodes, feat = x.shape

    def body(p0_ref, p1_ref, x_ref, wc_ref, b_ref, o_ref):
        agg = p0_ref[0] + p1_ref[0]
        xa = jnp.concatenate([agg, x_ref[...]], axis=-1)
        h = jnp.dot(xa, wc_ref[...], preferred_element_type=jnp.float32)
        o_ref[...] = jnp.maximum(h + b_ref[...], 0.0)

    grid = (n_nodes // blk,)
    p0_spec = pl.BlockSpec((1, blk, feat), lambda i: (0, i, 0))
    p1_spec = pl.BlockSpec((1, blk, feat), lambda i: (1, i, 0))
    row_spec = pl.BlockSpec((blk, feat), lambda i: (i, 0))
    wc_spec = pl.BlockSpec((2 * feat, feat), lambda i: (0, 0))
    bias_spec = pl.BlockSpec((1, feat), lambda i: (0, 0))
    return pl.pallas_call(
        body,
        grid=grid,
        in_specs=[p0_spec, p1_spec, row_spec, wc_spec, bias_spec],
        out_specs=row_spec,
        out_shape=jax.ShapeDtypeStruct((n_nodes, feat), jnp.float32),
    )(partials, partials, x, w_cat, bias2d)


def kernel(x, edge_index, W_self, W_rel, bias):
    n_nodes, feat = x.shape
    n_edges = edge_index.shape[1]

    per_w = -(-n_edges // NW)
    n_chunks = -(-(-(-per_w // CHUNK)) // 8) * 8
    padded = NW * n_chunks * CHUNK

    acc_rows = -(-(n_nodes + 1) // (NUM_SUBCORES * 8)) * (NUM_SUBCORES * 8)

    ei = edge_index.astype(jnp.int32)
    pad = jnp.tile(jnp.array([[0], [n_nodes]], jnp.int32), (1, padded - n_edges))
    e3 = jnp.concatenate([ei, pad], axis=1).reshape(2, NW * n_chunks, CHUNK)

    partials = _sc_segment_sum(x, e3, acc_rows, n_chunks)

    blk = 2000
    bias2d = bias.reshape(1, feat)
    w_cat = jnp.concatenate([W_rel, W_self], axis=0)
    return _tc_finish(partials, x, w_cat, bias2d, blk)


# speedup vs baseline: 2.3234x; 2.3234x over previous
import functools

import jax
import jax.numpy as jnp
from jax import lax
from jax.experimental import pallas as pl
from jax.experimental.pallas import tpu as pltpu
from jax.experimental.pallas import tpu_sc as plsc

CHUNK = 125
NUM_CORES = 2
NUM_SUBCORES = 16
NW = NUM_CORES * NUM_SUBCORES


def _sc_segment_sum(x, src3, dst3, acc_rows, n_chunks):
    n_nodes, feat = x.shape
    rpt = acc_rows // NUM_SUBCORES
    lanes = feat // 16

    mesh = plsc.VectorSubcoreMesh(core_axis_name="c", subcore_axis_name="s")

    @functools.partial(
        pl.kernel,
        mesh=mesh,
        out_type=jax.ShapeDtypeStruct((NUM_CORES, acc_rows, feat), jnp.float32),
        scratch_types=[
            pltpu.VMEM((n_chunks // 2, CHUNK), jnp.int32),
            pltpu.VMEM((n_chunks // 2, CHUNK), jnp.int32),
            [pltpu.VMEM((CHUNK, feat), jnp.float32) for _ in range(2)],
            pltpu.VMEM_SHARED((acc_rows, feat), jnp.float32),
            [pltpu.SemaphoreType.DMA for _ in range(2)],
            [pltpu.SemaphoreType.DMA for _ in range(2)],
        ],
    )
    def seg_sum(x_hbm, src_hbm, dst_hbm, out_hbm,
                src_v, dst_v, bufs, acc_sh, sem_g, sem_s):
        rows_v = bufs[0]
        c = lax.axis_index("c")
        s = lax.axis_index("s")
        wid = c * NUM_SUBCORES + s

        zrows = 120
        zero_v = jnp.zeros((16,), jnp.float32)

        def zero_row(r, carry):
            for l in range(lanes):
                rows_v[r, pl.ds(l * 16, 16)] = zero_v
            return carry

        lax.fori_loop(0, zrows, zero_row, 0)
        base = s * rpt
        off = 0
        while off < rpt:
            n = min(zrows, rpt - off)
            pltpu.sync_copy(rows_v.at[pl.ds(0, n)],
                            acc_sh.at[pl.ds(base + off, n)])
            off += n

        plsc.subcore_barrier()

        ph_chunks = n_chunks // 2
        n_pairs = ph_chunks // 2
        for phase in range(2):
            pltpu.sync_copy(
                src_hbm.at[wid, pl.ds(phase * ph_chunks, ph_chunks)], src_v)
            pltpu.sync_copy(
                dst_hbm.at[wid, pl.ds(phase * ph_chunks, ph_chunks)], dst_v)
            for b in range(2):
                pltpu.async_copy(x_hbm.at[src_v.at[b]], bufs[b], sem_g[b])

            def pair_body(i, carry):
                for b in range(2):
                    j = 2 * i + b
                    pltpu.make_async_copy(x_hbm.at[src_v.at[j]], bufs[b],
                                          sem_g[b]).wait()
                    pltpu.async_copy(bufs[b], acc_sh.at[dst_v.at[j]],
                                     sem_s[b], add=True)
                for b in range(2):
                    j = 2 * i + b
                    pltpu.make_async_copy(bufs[b], acc_sh.at[dst_v.at[j]],
                                          sem_s[b]).wait()

                    @pl.when(i < n_pairs - 1)
                    def _next_gather():
                        pltpu.async_copy(x_hbm.at[src_v.at[j + 2]], bufs[b],
                                         sem_g[b])
                return carry

            lax.fori_loop(0, n_pairs, pair_body, 0)
        plsc.subcore_barrier()
        pltpu.sync_copy(acc_sh.at[pl.ds(base, rpt)],
                        out_hbm.at[c, pl.ds(base, rpt)])

    return seg_sum(x, src3, dst3)


def _tc_finish(partials, x, w_cat, bias2d, blk):
    n_nodes, feat = x.shape

    def body(p0_ref, p1_ref, x_ref, wc_ref, b_ref, o_ref):
        agg = p0_ref[0] + p1_ref[0]
        xa = jnp.concatenate([agg, x_ref[...]], axis=-1)
        h = jnp.dot(xa, wc_ref[...], preferred_element_type=jnp.float32)
        o_ref[...] = jnp.maximum(h + b_ref[...], 0.0)

    grid = (n_nodes // blk,)
    p0_spec = pl.BlockSpec((1, blk, feat), lambda i: (0, i, 0))
    p1_spec = pl.BlockSpec((1, blk, feat), lambda i: (1, i, 0))
    row_spec = pl.BlockSpec((blk, feat), lambda i: (i, 0))
    wc_spec = pl.BlockSpec((2 * feat, feat), lambda i: (0, 0))
    bias_spec = pl.BlockSpec((1, feat), lambda i: (0, 0))
    return pl.pallas_call(
        body,
        grid=grid,
        in_specs=[p0_spec, p1_spec, row_spec, wc_spec, bias_spec],
        out_specs=row_spec,
        out_shape=jax.ShapeDtypeStruct((n_nodes, feat), jnp.float32),
    )(partials, partials, x, w_cat, bias2d)


def kernel(x, edge_index, W_self, W_rel, bias):
    n_nodes, feat = x.shape
    n_edges = edge_index.shape[1]

    per_w = n_edges // NW
    n_chunks = per_w // CHUNK
    assert per_w * NW == n_edges and n_chunks * CHUNK == per_w
    src3 = edge_index[0].astype(jnp.int32).reshape(NW, n_chunks, CHUNK)
    dst3 = edge_index[1].astype(jnp.int32).reshape(NW, n_chunks, CHUNK)

    acc_rows = -(-n_nodes // (NUM_SUBCORES * 8)) * (NUM_SUBCORES * 8)

    partials = _sc_segment_sum(x, src3, dst3, acc_rows, n_chunks)

    blk = 2000
    bias2d = bias.reshape(1, feat)
    w_cat = jnp.concatenate([W_rel, W_self], axis=0)
    return _tc_finish(partials, x, w_cat, bias2d, blk)


# speedup vs baseline: 2.4586x; 1.0582x over previous
import functools

import jax
import jax.numpy as jnp
from jax import lax
from jax.experimental import pallas as pl
from jax.experimental.pallas import tpu as pltpu
from jax.experimental.pallas import tpu_sc as plsc

CHUNK = 125
NUM_CORES = 2
NUM_SUBCORES = 16
NW = NUM_CORES * NUM_SUBCORES


def _sc_segment_sum(x, e4, acc_rows, n_chunks):
    n_nodes, feat = x.shape
    rpt = acc_rows // NUM_SUBCORES
    lanes = feat // 16

    mesh = plsc.VectorSubcoreMesh(core_axis_name="c", subcore_axis_name="s")

    @functools.partial(
        pl.kernel,
        mesh=mesh,
        out_type=jax.ShapeDtypeStruct((NUM_CORES, acc_rows, feat), jnp.float32),
        scratch_types=[
            pltpu.VMEM((n_chunks // 2, CHUNK), jnp.int32),
            pltpu.VMEM((n_chunks // 2, CHUNK), jnp.int32),
            [pltpu.VMEM((CHUNK, feat), jnp.float32) for _ in range(2)],
            pltpu.VMEM_SHARED((acc_rows, feat), jnp.float32),
            [pltpu.SemaphoreType.DMA for _ in range(2)],
            [pltpu.SemaphoreType.DMA for _ in range(2)],
        ],
    )
    def seg_sum(x_hbm, e4_hbm, out_hbm,
                src_v, dst_v, bufs, acc_sh, sem_g, sem_s):
        rows_v = bufs[0]
        c = lax.axis_index("c")
        s = lax.axis_index("s")
        wid = c * NUM_SUBCORES + s

        zrows = 120
        zero_v = jnp.zeros((16,), jnp.float32)

        def zero_row(r, carry):
            for l in range(lanes):
                rows_v[r, pl.ds(l * 16, 16)] = zero_v
            return carry

        lax.fori_loop(0, zrows, zero_row, 0)
        base = s * rpt
        off = 0
        while off < rpt:
            n = min(zrows, rpt - off)
            pltpu.sync_copy(rows_v.at[pl.ds(0, n)],
                            acc_sh.at[pl.ds(base + off, n)])
            off += n

        plsc.subcore_barrier()

        ph_chunks = n_chunks // 2
        n_pairs = ph_chunks // 2
        for phase in range(2):
            pltpu.sync_copy(
                e4_hbm.at[0, wid, pl.ds(phase * ph_chunks, ph_chunks)], src_v)
            pltpu.sync_copy(
                e4_hbm.at[1, wid, pl.ds(phase * ph_chunks, ph_chunks)], dst_v)
            for b in range(2):
                pltpu.async_copy(x_hbm.at[src_v.at[b]], bufs[b], sem_g[b])

            def pair_body(i, carry):
                for b in range(2):
                    j = 2 * i + b
                    pltpu.make_async_copy(x_hbm.at[src_v.at[j]], bufs[b],
                                          sem_g[b]).wait()
                    pltpu.async_copy(bufs[b], acc_sh.at[dst_v.at[j]],
                                     sem_s[b], add=True)
                for b in range(2):
                    j = 2 * i + b
                    pltpu.make_async_copy(bufs[b], acc_sh.at[dst_v.at[j]],
                                          sem_s[b]).wait()

                    @pl.when(i < n_pairs - 1)
                    def _next_gather():
                        pltpu.async_copy(x_hbm.at[src_v.at[j + 2]], bufs[b],
                                         sem_g[b])
                return carry

            lax.fori_loop(0, n_pairs, pair_body, 0)
        plsc.subcore_barrier()
        pltpu.sync_copy(acc_sh.at[pl.ds(base, rpt)],
                        out_hbm.at[c, pl.ds(base, rpt)])

    return seg_sum(x, e4)


def _tc_finish(partials, x, w_cat, bias2d, blk):
    n_nodes, feat = x.shape

    def body(p0_ref, p1_ref, x_ref, wc_ref, b_ref, o_ref):
        agg = p0_ref[0] + p1_ref[0]
        xa = jnp.concatenate([agg, x_ref[...]], axis=-1)
        h = jnp.dot(xa, wc_ref[...], preferred_element_type=jnp.float32)
        o_ref[...] = jnp.maximum(h + b_ref[...], 0.0)

    grid = (n_nodes // blk,)
    p0_spec = pl.BlockSpec((1, blk, feat), lambda i: (0, i, 0))
    p1_spec = pl.BlockSpec((1, blk, feat), lambda i: (1, i, 0))
    row_spec = pl.BlockSpec((blk, feat), lambda i: (i, 0))
    wc_spec = pl.BlockSpec((2 * feat, feat), lambda i: (0, 0))
    bias_spec = pl.BlockSpec((1, feat), lambda i: (0, 0))
    return pl.pallas_call(
        body,
        grid=grid,
        in_specs=[p0_spec, p1_spec, row_spec, wc_spec, bias_spec],
        out_specs=row_spec,
        out_shape=jax.ShapeDtypeStruct((n_nodes, feat), jnp.float32),
    )(partials, partials, x, w_cat, bias2d)


def kernel(x, edge_index, W_self, W_rel, bias):
    n_nodes, feat = x.shape
    n_edges = edge_index.shape[1]

    per_w = n_edges // NW
    n_chunks = per_w // CHUNK
    assert per_w * NW == n_edges and n_chunks * CHUNK == per_w
    e4 = edge_index.astype(jnp.int32).reshape(2, NW, n_chunks, CHUNK)

    acc_rows = -(-n_nodes // (NUM_SUBCORES * 8)) * (NUM_SUBCORES * 8)

    partials = _sc_segment_sum(x, e4, acc_rows, n_chunks)

    blk = 2000
    bias2d = bias.reshape(1, feat)
    w_cat = jnp.concatenate([W_rel, W_self], axis=0)
    return _tc_finish(partials, x, w_cat, bias2d, blk)
